# baseline (speedup 1.0000x reference)
import functools

import jax
import jax.numpy as jnp
from jax import lax
from jax.experimental import pallas as pl
from jax.experimental.pallas import tpu as pltpu
from jax.experimental.pallas import tpu_sc as plsc

_NC = 2
_NS = 16
_NW = _NC * _NS
_LANES = 16


def _pick_chunk(per_worker: int, max_k: int) -> int:
    for k in range(min(max_k, 128) // 8 * 8, 0, -8):
        if per_worker % k == 0:
            return k
    raise ValueError(f"no valid chunk size for per_worker={per_worker}")


def _sc_aggregate(ego, adj, src, dst):
    N, D = ego.shape
    E = adj.shape[0]
    assert D % _LANES == 0
    assert E % _NW == 0
    per_worker = E // _NW
    max_k = (51000 - 2 * per_worker) // (5 * (D + 1))
    K = _pick_chunk(per_worker, max_k)
    nchunks = per_worker // K
    assert N % K == 0
    nblocks = N // K
    blk_full, blk_rem = divmod(nblocks, _NS)
    fgroups = D // _LANES

    mesh = plsc.VectorSubcoreMesh(
        core_axis_name="c", subcore_axis_name="s",
        num_cores=_NC, num_subcores=_NS,
    )

    assert nchunks >= 6

    @functools.partial(
        pl.kernel,
        out_type=jax.ShapeDtypeStruct((_NC, N, D), jnp.float32),
        mesh=mesh,
        compiler_params=pltpu.CompilerParams(needs_layout_passes=False),
        scratch_types=[
            pltpu.VMEM_SHARED((N, D), jnp.float32),
            pltpu.VMEM((per_worker,), jnp.int32),
            pltpu.VMEM((per_worker,), jnp.float32),
            pltpu.VMEM((5, K), jnp.int32),
            pltpu.VMEM((5, K, D), jnp.float32),
        ] + [pltpu.SemaphoreType.DMA] * 15,
    )
    def agg(ego_hbm, adj_hbm, src_hbm, dst_hbm, out_hbm,
            accum, src_all, aval_all, dstb, rows, *sems):
        gsems = sems[0:5]
        dsems = sems[5:10]
        ssems = sems[10:15]
        c = lax.axis_index("c")
        s = lax.axis_index("s")
        wid = c * _NS + s
        ebase = wid * per_worker

        def zfill(i, _):
            for j in range(fgroups):
                rows[0, i, pl.ds(j * _LANES, _LANES)] = jnp.zeros(
                    (_LANES,), jnp.float32)
            return 0
        lax.fori_loop(0, K, zfill, 0)
        my_blocks = jnp.where(s < blk_rem, blk_full + 1, blk_full)

        def zcopy(i, _):
            b = s + i * _NS
            pltpu.sync_copy(rows.at[0], accum.at[pl.ds(b * K, K)])
            return 0
        lax.fori_loop(0, my_blocks, zcopy, 0)
        plsc.subcore_barrier()

        def src_slice(ci):
            return src_all.at[pl.ds(ci * K, K)]

        def issue_gather(ci, p):
            pltpu.async_copy(ego_hbm.at[src_slice(ci)], rows.at[p], gsems[p])

        def wait_gather(ci, p):
            pltpu.make_async_copy(
                ego_hbm.at[src_slice(ci)], rows.at[p], gsems[p]).wait()

        def issue_dst(ci, p):
            pltpu.async_copy(
                dst_hbm.at[pl.ds(ebase + ci * K, K)], dstb.at[p], dsems[p])

        def wait_dst(ci, p):
            pltpu.make_async_copy(
                dst_hbm.at[pl.ds(ebase + ci * K, K)], dstb.at[p],
                dsems[p]).wait()

        def issue_scatter(p):
            pltpu.async_copy(rows.at[p], accum.at[dstb.at[p]], ssems[p],
                             add=True)

        def wait_scatter(p):
            pltpu.make_async_copy(
                rows.at[p], accum.at[dstb.at[p]], ssems[p]).wait()

        def scale(ci, p):
            cbase = ci * K

            unroll = 4

            def scale_body(i, _):
                es = [unroll * i + u for u in range(unroll)]
                abs_ = [plsc.load_gather(
                    aval_all, [jnp.full((_LANES,), cbase + e, jnp.int32)])
                    for e in es]
                for j in range(fgroups):
                    sl = pl.ds(j * _LANES, _LANES)
                    for e, ab in zip(es, abs_):
                        rows[p, e, sl] = rows[p, e, sl] * ab
                return 0
            lax.fori_loop(0, K // unroll, scale_body, 0)

        pltpu.sync_copy(src_hbm.at[pl.ds(ebase, per_worker)], src_all)
        pltpu.sync_copy(adj_hbm.at[pl.ds(ebase, per_worker)], aval_all)
        for ci0 in range(2):
            issue_dst(ci0, ci0)
            issue_gather(ci0, ci0)

        def super_body(t, _):
            for b in range(5):
                ci = 5 * t + b
                p = b
                pw = (b + 2) % 5
                @pl.when(ci >= 3)
                def _():
                    wait_scatter(pw)
                @pl.when(ci + 2 < nchunks)
                def _():
                    issue_dst(ci + 2, pw)
                    issue_gather(ci + 2, pw)
                wait_gather(ci, p)
                scale(ci, p)
                wait_dst(ci, p)
                issue_scatter(p)
            return 0
        nq = nchunks // 5
        lax.fori_loop(0, nq, super_body, 0)

        for ci in range(5 * nq, nchunks):
            p = ci % 5
            wait_scatter((ci + 2) % 5)
            wait_gather(ci, p)
            scale(ci, p)
            wait_dst(ci, p)
            issue_scatter(p)
        wait_scatter((nchunks - 3) % 5)
        wait_scatter((nchunks - 2) % 5)
        wait_scatter((nchunks - 1) % 5)

        plsc.subcore_barrier()

        def ocopy(i, _):
            b = s + i * _NS
            sl = pl.ds(b * K, K)
            pltpu.sync_copy(accum.at[sl], out_hbm.at[c].at[sl])
            return 0
        lax.fori_loop(0, my_blocks, ocopy, 0)

    return agg(ego, adj, src, dst)


def _tc_matmul(x, W):
    N, D = x.shape
    BM = 1000
    assert N % BM == 0

    def body(x_ref, w_ref, out_ref):
        out_ref[...] = jnp.dot(x_ref[...], w_ref[...],
                               preferred_element_type=jnp.float32)

    row_spec = pl.BlockSpec((BM, D), lambda i: (i, 0))
    w_spec = pl.BlockSpec((D, D), lambda i: (0, 0))
    return pl.pallas_call(
        body,
        grid=(N // BM,),
        in_specs=[row_spec, w_spec],
        out_specs=row_spec,
        out_shape=jax.ShapeDtypeStruct((N, D), jnp.float32),
    )(x, W)


def _tc_tail(p0, p1, sp, W2):
    N, D = sp.shape
    BM = 1000
    assert N % BM == 0

    def body(p0_ref, p1_ref, sp_ref, w2_ref, out_ref):
        nb = p0_ref[...] + p1_ref[...]
        sp = sp_ref[...]
        npart = jnp.dot(nb, w2_ref[...],
                        preferred_element_type=jnp.float32)
        y = sp + npart + sp * npart
        out_ref[...] = jnp.where(y >= 0, y, 0.2 * y)

    row_spec = pl.BlockSpec((BM, D), lambda i: (i, 0))
    w_spec = pl.BlockSpec((D, D), lambda i: (0, 0))
    return pl.pallas_call(
        body,
        grid=(N // BM,),
        in_specs=[row_spec, row_spec, row_spec, w_spec],
        out_specs=row_spec,
        out_shape=jax.ShapeDtypeStruct((N, D), jnp.float32),
    )(p0, p1, sp, W2)


@jax.jit
def kernel(ego_embeddings, adj_values, W1, W2, edge_index):
    src = edge_index[0]
    dst = edge_index[1]
    partials = _sc_aggregate(ego_embeddings, adj_values, src, dst)
    sp = _tc_matmul(ego_embeddings, W1)
    return _tc_tail(partials[0], partials[1], sp, W2)


# speedup vs baseline: 1.0087x; 1.0087x over previous
import functools

import jax
import jax.numpy as jnp
from jax import lax
from jax.experimental import pallas as pl
from jax.experimental.pallas import tpu as pltpu
from jax.experimental.pallas import tpu_sc as plsc

_NC = 2
_NS = 16
_NW = _NC * _NS
_LANES = 16


def _pick_chunk(per_worker: int, max_k: int) -> int:
    for k in range(min(max_k, 128) // 8 * 8, 0, -8):
        if per_worker % k == 0:
            return k
    raise ValueError(f"no valid chunk size for per_worker={per_worker}")


def _sc_aggregate(ego, adj, src, dst):
    N, D = ego.shape
    E = adj.shape[0]
    assert D % _LANES == 0
    assert E % _NW == 0
    per_worker = E // _NW
    max_k = (50000 - 2 * per_worker) // (4 * (D + 1))
    K = _pick_chunk(per_worker, max_k)
    nchunks = per_worker // K
    assert N % K == 0
    nblocks = N // K
    blk_full, blk_rem = divmod(nblocks, _NS)
    fgroups = D // _LANES

    mesh = plsc.VectorSubcoreMesh(
        core_axis_name="c", subcore_axis_name="s",
        num_cores=_NC, num_subcores=_NS,
    )

    assert nchunks >= 6

    @functools.partial(
        pl.kernel,
        out_type=jax.ShapeDtypeStruct((_NC, N, D), jnp.float32),
        mesh=mesh,
        compiler_params=pltpu.CompilerParams(needs_layout_passes=False),
        scratch_types=[
            pltpu.VMEM_SHARED((N, D), jnp.float32),
            pltpu.VMEM((per_worker,), jnp.int32),
            pltpu.VMEM((per_worker,), jnp.float32),
            pltpu.VMEM((4, K), jnp.int32),
            pltpu.VMEM((4, K, D), jnp.float32),
        ] + [pltpu.SemaphoreType.DMA] * 12,
    )
    def agg(ego_hbm, adj_hbm, src_hbm, dst_hbm, out_hbm,
            accum, src_all, aval_all, dstb, rows, *sems):
        gsems = sems[0:4]
        dsems = sems[4:8]
        ssems = sems[8:12]
        c = lax.axis_index("c")
        s = lax.axis_index("s")
        wid = c * _NS + s
        ebase = wid * per_worker

        def zfill(i, _):
            for j in range(fgroups):
                rows[0, i, pl.ds(j * _LANES, _LANES)] = jnp.zeros(
                    (_LANES,), jnp.float32)
            return 0
        lax.fori_loop(0, K, zfill, 0)
        my_blocks = jnp.where(s < blk_rem, blk_full + 1, blk_full)

        def zcopy(i, _):
            b = s + i * _NS
            pltpu.sync_copy(rows.at[0], accum.at[pl.ds(b * K, K)])
            return 0
        lax.fori_loop(0, my_blocks, zcopy, 0)
        plsc.subcore_barrier()

        def src_slice(ci):
            return src_all.at[pl.ds(ci * K, K)]

        def issue_gather(ci, p):
            pltpu.async_copy(ego_hbm.at[src_slice(ci)], rows.at[p], gsems[p])

        def wait_gather(ci, p):
            pltpu.make_async_copy(
                ego_hbm.at[src_slice(ci)], rows.at[p], gsems[p]).wait()

        def issue_dst(ci, p):
            pltpu.async_copy(
                dst_hbm.at[pl.ds(ebase + ci * K, K)], dstb.at[p], dsems[p])

        def wait_dst(ci, p):
            pltpu.make_async_copy(
                dst_hbm.at[pl.ds(ebase + ci * K, K)], dstb.at[p],
                dsems[p]).wait()

        def issue_scatter(p):
            pltpu.async_copy(rows.at[p], accum.at[dstb.at[p]], ssems[p],
                             add=True)

        def wait_scatter(p):
            pltpu.make_async_copy(
                rows.at[p], accum.at[dstb.at[p]], ssems[p]).wait()

        def scale(ci, p):
            cbase = ci * K

            unroll = 4

            def scale_body(i, _):
                es = [unroll * i + u for u in range(unroll)]
                abs_ = [plsc.load_gather(
                    aval_all, [jnp.full((_LANES,), cbase + e, jnp.int32)])
                    for e in es]
                for j in range(fgroups):
                    sl = pl.ds(j * _LANES, _LANES)
                    for e, ab in zip(es, abs_):
                        rows[p, e, sl] = rows[p, e, sl] * ab
                return 0
            lax.fori_loop(0, K // unroll, scale_body, 0)

        pltpu.sync_copy(src_hbm.at[pl.ds(ebase, per_worker)], src_all)
        pltpu.sync_copy(adj_hbm.at[pl.ds(ebase, per_worker)], aval_all)
        for ci0 in range(2):
            issue_dst(ci0, ci0)
            issue_gather(ci0, ci0)

        def super_body(t, _):
            for b in range(4):
                ci = 4 * t + b
                p = b
                pw = (b + 2) % 4
                @pl.when(ci >= 2)
                def _():
                    wait_scatter(pw)
                @pl.when(ci + 2 < nchunks)
                def _():
                    issue_dst(ci + 2, pw)
                    issue_gather(ci + 2, pw)
                wait_gather(ci, p)
                scale(ci, p)
                wait_dst(ci, p)
                issue_scatter(p)
            return 0
        nq = nchunks // 4
        lax.fori_loop(0, nq, super_body, 0)

        for ci in range(4 * nq, nchunks):
            p = ci % 4
            wait_scatter((ci + 2) % 4)
            wait_gather(ci, p)
            scale(ci, p)
            wait_dst(ci, p)
            issue_scatter(p)
        wait_scatter((nchunks - 2) % 4)
        wait_scatter((nchunks - 1) % 4)

        plsc.subcore_barrier()

        def ocopy(i, _):
            b = s + i * _NS
            sl = pl.ds(b * K, K)
            pltpu.sync_copy(accum.at[sl], out_hbm.at[c].at[sl])
            return 0
        lax.fori_loop(0, my_blocks, ocopy, 0)

    return agg(ego, adj, src, dst)


def _tc_matmul(x, W):
    N, D = x.shape
    BM = 1000
    assert N % BM == 0

    def body(x_ref, w_ref, out_ref):
        out_ref[...] = jnp.dot(x_ref[...], w_ref[...],
                               preferred_element_type=jnp.float32)

    row_spec = pl.BlockSpec((BM, D), lambda i: (i, 0))
    w_spec = pl.BlockSpec((D, D), lambda i: (0, 0))
    return pl.pallas_call(
        body,
        grid=(N // BM,),
        in_specs=[row_spec, w_spec],
        out_specs=row_spec,
        out_shape=jax.ShapeDtypeStruct((N, D), jnp.float32),
    )(x, W)


def _tc_tail(p0, p1, sp, W2):
    N, D = sp.shape
    BM = 1000
    assert N % BM == 0

    def body(p0_ref, p1_ref, sp_ref, w2_ref, out_ref):
        nb = p0_ref[...] + p1_ref[...]
        sp = sp_ref[...]
        npart = jnp.dot(nb, w2_ref[...],
                        preferred_element_type=jnp.float32)
        y = sp + npart + sp * npart
        out_ref[...] = jnp.where(y >= 0, y, 0.2 * y)

    row_spec = pl.BlockSpec((BM, D), lambda i: (i, 0))
    w_spec = pl.BlockSpec((D, D), lambda i: (0, 0))
    return pl.pallas_call(
        body,
        grid=(N // BM,),
        in_specs=[row_spec, row_spec, row_spec, w_spec],
        out_specs=row_spec,
        out_shape=jax.ShapeDtypeStruct((N, D), jnp.float32),
    )(p0, p1, sp, W2)


@jax.jit
def kernel(ego_embeddings, adj_values, W1, W2, edge_index):
    src = edge_index[0]
    dst = edge_index[1]
    partials = _sc_aggregate(ego_embeddings, adj_values, src, dst)
    sp = _tc_matmul(ego_embeddings, W1)
    return _tc_tail(partials[0], partials[1], sp, W2)


# speedup vs baseline: 1.0362x; 1.0273x over previous
import functools

import jax
import jax.numpy as jnp
from jax import lax
from jax.experimental import pallas as pl
from jax.experimental.pallas import tpu as pltpu
from jax.experimental.pallas import tpu_sc as plsc

_NC = 2
_NS = 16
_NW = _NC * _NS
_LANES = 16


def _pick_chunk(per_worker: int, max_k: int) -> int:
    for k in range(min(max_k, 128) // 8 * 8, 0, -8):
        if per_worker % k == 0:
            return k
    raise ValueError(f"no valid chunk size for per_worker={per_worker}")


def _sc_aggregate(ego, adj, src, dst):
    N, D = ego.shape
    E = adj.shape[0]
    assert D % _LANES == 0
    assert E % _NW == 0
    per_worker = E // _NW
    max_k = (50000 - 2 * per_worker) // (4 * (D + 1))
    K = _pick_chunk(per_worker, max_k)
    nchunks = per_worker // K
    assert N % K == 0
    nblocks = N // K
    blk_full, blk_rem = divmod(nblocks, _NS)
    fgroups = D // _LANES

    mesh = plsc.VectorSubcoreMesh(
        core_axis_name="c", subcore_axis_name="s",
        num_cores=_NC, num_subcores=_NS,
    )

    assert nchunks >= 6

    @functools.partial(
        pl.kernel,
        out_type=jax.ShapeDtypeStruct((_NC, N, D), jnp.float32),
        mesh=mesh,
        compiler_params=pltpu.CompilerParams(needs_layout_passes=False),
        scratch_types=[
            pltpu.VMEM_SHARED((N, D), jnp.float32),
            pltpu.VMEM((per_worker,), jnp.int32),
            pltpu.VMEM((per_worker,), jnp.float32),
            pltpu.VMEM((4, K), jnp.int32),
            pltpu.VMEM((4, K, D), jnp.float32),
        ] + [pltpu.SemaphoreType.DMA] * 12,
    )
    def agg(ego_hbm, adj_hbm, src_hbm, dst_hbm, out_hbm,
            accum, src_all, aval_all, dstb, rows, *sems):
        gsems = sems[0:4]
        dsems = sems[4:8]
        ssems = sems[8:12]
        c = lax.axis_index("c")
        s = lax.axis_index("s")
        wid = c * _NS + s
        ebase = wid * per_worker

        def zfill(i, _):
            for j in range(fgroups):
                rows[0, i, pl.ds(j * _LANES, _LANES)] = jnp.zeros(
                    (_LANES,), jnp.float32)
            return 0
        lax.fori_loop(0, K, zfill, 0)
        my_blocks = jnp.where(s < blk_rem, blk_full + 1, blk_full)

        def zcopy(i, _):
            b = s + i * _NS
            pltpu.async_copy(rows.at[0], accum.at[pl.ds(b * K, K)], sems[0])
            return 0
        lax.fori_loop(0, my_blocks, zcopy, 0)

        def zdrain(i, _):
            b = s + i * _NS
            pltpu.make_async_copy(
                rows.at[0], accum.at[pl.ds(b * K, K)], sems[0]).wait()
            return 0
        lax.fori_loop(0, my_blocks, zdrain, 0)
        plsc.subcore_barrier()

        def src_slice(ci):
            return src_all.at[pl.ds(ci * K, K)]

        def issue_gather(ci, p):
            pltpu.async_copy(ego_hbm.at[src_slice(ci)], rows.at[p], gsems[p])

        def wait_gather(ci, p):
            pltpu.make_async_copy(
                ego_hbm.at[src_slice(ci)], rows.at[p], gsems[p]).wait()

        def issue_dst(ci, p):
            pltpu.async_copy(
                dst_hbm.at[pl.ds(ebase + ci * K, K)], dstb.at[p], dsems[p])

        def wait_dst(ci, p):
            pltpu.make_async_copy(
                dst_hbm.at[pl.ds(ebase + ci * K, K)], dstb.at[p],
                dsems[p]).wait()

        def issue_scatter(p):
            pltpu.async_copy(rows.at[p], accum.at[dstb.at[p]], ssems[p],
                             add=True)

        def wait_scatter(p):
            pltpu.make_async_copy(
                rows.at[p], accum.at[dstb.at[p]], ssems[p]).wait()

        def scale(ci, p):
            cbase = ci * K

            unroll = 4

            def scale_body(i, _):
                es = [unroll * i + u for u in range(unroll)]
                abs_ = [plsc.load_gather(
                    aval_all, [jnp.full((_LANES,), cbase + e, jnp.int32)])
                    for e in es]
                for j in range(fgroups):
                    sl = pl.ds(j * _LANES, _LANES)
                    for e, ab in zip(es, abs_):
                        rows[p, e, sl] = rows[p, e, sl] * ab
                return 0
            lax.fori_loop(0, K // unroll, scale_body, 0)

        pltpu.sync_copy(src_hbm.at[pl.ds(ebase, per_worker)], src_all)
        pltpu.sync_copy(adj_hbm.at[pl.ds(ebase, per_worker)], aval_all)
        for ci0 in range(2):
            issue_dst(ci0, ci0)
            issue_gather(ci0, ci0)

        def super_body(t, _):
            for b in range(4):
                ci = 4 * t + b
                p = b
                pw = (b + 2) % 4
                @pl.when(ci >= 2)
                def _():
                    wait_scatter(pw)
                @pl.when(ci + 2 < nchunks)
                def _():
                    issue_dst(ci + 2, pw)
                    issue_gather(ci + 2, pw)
                wait_gather(ci, p)
                scale(ci, p)
                wait_dst(ci, p)
                issue_scatter(p)
            return 0
        nq = nchunks // 4
        lax.fori_loop(0, nq, super_body, 0)

        for ci in range(4 * nq, nchunks):
            p = ci % 4
            wait_scatter((ci + 2) % 4)
            wait_gather(ci, p)
            scale(ci, p)
            wait_dst(ci, p)
            issue_scatter(p)
        wait_scatter((nchunks - 2) % 4)
        wait_scatter((nchunks - 1) % 4)

        plsc.subcore_barrier()

        def ocopy(i, _):
            b = s + i * _NS
            sl = pl.ds(b * K, K)
            pltpu.async_copy(accum.at[sl], out_hbm.at[c].at[sl], sems[0])
            return 0
        lax.fori_loop(0, my_blocks, ocopy, 0)

        def odrain(i, _):
            b = s + i * _NS
            sl = pl.ds(b * K, K)
            pltpu.make_async_copy(
                accum.at[sl], out_hbm.at[c].at[sl], sems[0]).wait()
            return 0
        lax.fori_loop(0, my_blocks, odrain, 0)

    return agg(ego, adj, src, dst)


def _tc_matmul(x, W):
    N, D = x.shape
    BM = 1000
    assert N % BM == 0

    def body(x_ref, w_ref, out_ref):
        out_ref[...] = jnp.dot(x_ref[...], w_ref[...],
                               preferred_element_type=jnp.float32)

    row_spec = pl.BlockSpec((BM, D), lambda i: (i, 0))
    w_spec = pl.BlockSpec((D, D), lambda i: (0, 0))
    return pl.pallas_call(
        body,
        grid=(N // BM,),
        in_specs=[row_spec, w_spec],
        out_specs=row_spec,
        out_shape=jax.ShapeDtypeStruct((N, D), jnp.float32),
    )(x, W)


def _tc_tail(p0, p1, sp, W2):
    N, D = sp.shape
    BM = 1000
    assert N % BM == 0

    def body(p0_ref, p1_ref, sp_ref, w2_ref, out_ref):
        nb = p0_ref[...] + p1_ref[...]
        sp = sp_ref[...]
        npart = jnp.dot(nb, w2_ref[...],
                        preferred_element_type=jnp.float32)
        y = sp + npart + sp * npart
        out_ref[...] = jnp.where(y >= 0, y, 0.2 * y)

    row_spec = pl.BlockSpec((BM, D), lambda i: (i, 0))
    w_spec = pl.BlockSpec((D, D), lambda i: (0, 0))
    return pl.pallas_call(
        body,
        grid=(N // BM,),
        in_specs=[row_spec, row_spec, row_spec, w_spec],
        out_specs=row_spec,
        out_shape=jax.ShapeDtypeStruct((N, D), jnp.float32),
    )(p0, p1, sp, W2)


@jax.jit
def kernel(ego_embeddings, adj_values, W1, W2, edge_index):
    src = edge_index[0]
    dst = edge_index[1]
    partials = _sc_aggregate(ego_embeddings, adj_values, src, dst)
    sp = _tc_matmul(ego_embeddings, W1)
    return _tc_tail(partials[0], partials[1], sp, W2)
